# baseline (speedup 1.0000x reference)
import functools

import jax
import jax.numpy as jnp
from jax.experimental import pallas as pl
from jax.experimental.pallas import tpu as pltpu


def _pe_add_block(x_ref, time_ref, pos_ref, out_ref):
    out_ref[0] = x_ref[0] + (pos_ref[...] + time_ref[...])


@functools.partial(jax.jit, static_argnames=("block_l",))
def _pe_add(x, time_row, pos_emb, block_l):
    B, L, D = x.shape
    grid = (L // block_l, B)
    return pl.pallas_call(
        _pe_add_block,
        grid=grid,
        in_specs=[
            pl.BlockSpec((1, block_l, D), lambda l, b: (b, l, 0)),
            pl.BlockSpec((1, D), lambda l, b: (0, 0)),
            pl.BlockSpec((block_l, D), lambda l, b: (l, 0)),
        ],
        out_specs=pl.BlockSpec((1, block_l, D), lambda l, b: (b, l, 0)),
        out_shape=jax.ShapeDtypeStruct((B, L, D), x.dtype),
        compiler_params=pltpu.CompilerParams(
            vmem_limit_bytes=110 * 1024 * 1024,
        ),
    )(x, time_row, pos_emb)


def kernel(x, tgt_time_step, time_emb, pos_emb):
    t = jnp.asarray(tgt_time_step, jnp.int32) - 1
    time_row = jax.lax.dynamic_slice_in_dim(time_emb, t, 1, axis=0)
    return _pe_add(x, time_row, pos_emb, block_l=2048)


# speedup vs baseline: 1.0034x; 1.0034x over previous
import functools

import jax
import jax.numpy as jnp
from jax.experimental import pallas as pl
from jax.experimental.pallas import tpu as pltpu


def _pe_add_block(x_ref, time_ref, pos_ref, out_ref):
    out_ref[0] = x_ref[0] + (pos_ref[...] + time_ref[...])


@functools.partial(jax.jit, static_argnames=("block_l",))
def _pe_add(x, time_row, pos_emb, block_l):
    B, L, D = x.shape
    grid = (L // block_l, B)
    return pl.pallas_call(
        _pe_add_block,
        grid=grid,
        in_specs=[
            pl.BlockSpec((1, block_l, D), lambda l, b: (b, l, 0)),
            pl.BlockSpec((1, D), lambda l, b: (0, 0)),
            pl.BlockSpec((block_l, D), lambda l, b: (l, 0)),
        ],
        out_specs=pl.BlockSpec((1, block_l, D), lambda l, b: (b, l, 0)),
        out_shape=jax.ShapeDtypeStruct((B, L, D), x.dtype),
        compiler_params=pltpu.CompilerParams(
            dimension_semantics=("parallel", "parallel"),
            vmem_limit_bytes=110 * 1024 * 1024,
        ),
    )(x, time_row, pos_emb)


def kernel(x, tgt_time_step, time_emb, pos_emb):
    t = jnp.asarray(tgt_time_step, jnp.int32) - 1
    time_row = jax.lax.dynamic_slice_in_dim(time_emb, t, 1, axis=0)
    return _pe_add(x, time_row, pos_emb, block_l=2048)


# speedup vs baseline: 1.0164x; 1.0129x over previous
import functools

import jax
import jax.numpy as jnp
from jax.experimental import pallas as pl
from jax.experimental.pallas import tpu as pltpu


def _pe_add_block(x_ref, time_ref, pos_ref, out_ref):
    pe = pos_ref[...] + time_ref[...]
    out_ref[...] = x_ref[...] + pe[None, :, :]


@functools.partial(jax.jit, static_argnames=("block_l",))
def _pe_add(x, time_row, pos_emb, block_l):
    B, L, D = x.shape
    grid = (L // block_l,)
    return pl.pallas_call(
        _pe_add_block,
        grid=grid,
        in_specs=[
            pl.BlockSpec((B, block_l, D), lambda l: (0, l, 0)),
            pl.BlockSpec((1, D), lambda l: (0, 0)),
            pl.BlockSpec((block_l, D), lambda l: (l, 0)),
        ],
        out_specs=pl.BlockSpec((B, block_l, D), lambda l: (0, l, 0)),
        out_shape=jax.ShapeDtypeStruct((B, L, D), x.dtype),
        compiler_params=pltpu.CompilerParams(
            dimension_semantics=("arbitrary",),
        ),
    )(x, time_row, pos_emb)


def kernel(x, tgt_time_step, time_emb, pos_emb):
    t = jnp.asarray(tgt_time_step, jnp.int32) - 1
    time_row = jax.lax.dynamic_slice_in_dim(time_emb, t, 1, axis=0)
    return _pe_add(x, time_row, pos_emb, block_l=512)


# speedup vs baseline: 1.0168x; 1.0005x over previous
import functools

import jax
import jax.numpy as jnp
from jax.experimental import pallas as pl
from jax.experimental.pallas import tpu as pltpu


def _pe_add_block(x_ref, time_ref, pos_ref, out_ref):
    pe = pos_ref[...] + time_ref[...]
    out_ref[...] = x_ref[...] + pe[None, :, :]


@functools.partial(jax.jit, static_argnames=("block_l",))
def _pe_add(x, time_row, pos_emb, block_l):
    B, L, D = x.shape
    grid = (L // block_l,)
    return pl.pallas_call(
        _pe_add_block,
        grid=grid,
        in_specs=[
            pl.BlockSpec((B, block_l, D), lambda l: (0, l, 0)),
            pl.BlockSpec((1, D), lambda l: (0, 0)),
            pl.BlockSpec((block_l, D), lambda l: (l, 0)),
        ],
        out_specs=pl.BlockSpec((B, block_l, D), lambda l: (0, l, 0)),
        out_shape=jax.ShapeDtypeStruct((B, L, D), x.dtype),
        compiler_params=pltpu.CompilerParams(
            dimension_semantics=("arbitrary",),
        ),
    )(x, time_row, pos_emb)


def kernel(x, tgt_time_step, time_emb, pos_emb):
    t = jnp.asarray(tgt_time_step, jnp.int32) - 1
    time_row = jax.lax.dynamic_slice_in_dim(time_emb, t, 1, axis=0)
    return _pe_add(x, time_row, pos_emb, block_l=256)
